# baseline (speedup 1.0000x reference)
import functools

import jax
import jax.numpy as jnp
from jax import lax
from jax.experimental import pallas as pl
from jax.experimental.pallas import tpu as pltpu
from jax.experimental.pallas import tpu_sc as plsc

B = 4096
S = 200
D = 64
C = 2
V = 1000000
HALF = S // 2
HPAD = 104
GBUF = 2 * HPAD
NC = 2
NS = 16
NW = NC * NS
BPW = B // NW
INV_S = 1.0 / S
NB = 16384
NDEEP = 16


def _proj_body(w_ref, t_ref, p_ref):
    m = lax.dot_general(w_ref[...], t_ref[...], (((1,), (0,)), ((), ())),
                        preferred_element_type=jnp.float32)
    r0 = m[0].astype(jnp.bfloat16).astype(jnp.float32)
    r1 = m[1].astype(jnp.bfloat16).astype(jnp.float32)
    b0 = lax.shift_right_logical(lax.bitcast_convert_type(r0, jnp.int32), 16)
    b1 = jnp.bitwise_and(lax.bitcast_convert_type(r1, jnp.int32),
                         jnp.int32(-65536))
    p_ref[...] = jnp.bitwise_or(b0, b1)


def _project(head_W, table_t):
    return pl.pallas_call(
        _proj_body,
        grid=(pl.cdiv(V, NB),),
        in_specs=[
            pl.BlockSpec((C, D), lambda k: (0, 0)),
            pl.BlockSpec((D, NB), lambda k: (0, k)),
        ],
        out_specs=[
            pl.BlockSpec((NB,), lambda k: (k,)),
        ],
        out_shape=[
            jax.ShapeDtypeStruct((V,), jnp.int32),
        ],
    )(head_W, table_t)


def _sc_body(ids_hbm, p_hbm, b_hbm, out_hbm,
             ids_v, dbuf, b_v, out_v, *sems):
    wid = lax.axis_index("s") * NC + lax.axis_index("c")
    base = wid * BPW

    pltpu.sync_copy(ids_hbm.at[pl.ds(base * 2, BPW * 2)], ids_v)
    pltpu.sync_copy(b_hbm, b_v)

    def issue(e, k):
        pltpu.async_copy(p_hbm.at[ids_v.at[2 * e]],
                         dbuf.at[k, pl.ds(0, HPAD)], sems[k])
        pltpu.async_copy(p_hbm.at[ids_v.at[2 * e + 1]],
                         dbuf.at[k, pl.ds(HPAD, HPAD)], sems[k])

    def drain(e, k):
        pltpu.make_async_copy(p_hbm.at[ids_v.at[2 * e]],
                              dbuf.at[k, pl.ds(0, HPAD)], sems[k]).wait()
        pltpu.make_async_copy(p_hbm.at[ids_v.at[2 * e + 1]],
                              dbuf.at[k, pl.ds(HPAD, HPAD)], sems[k]).wait()

    for k in range(NDEEP):
        issue(k, k)

    lane = lax.iota(jnp.int32, 16)
    himask = jnp.full((16,), -65536, jnp.int32)

    @pl.loop(0, BPW, step=NDEEP)
    def _elements(e0):
        for k in range(NDEEP):
            e = e0 + k
            drain(e, k)

            acc0 = jnp.zeros((16,), jnp.float32)
            acc1 = jnp.zeros((16,), jnp.float32)
            for i in range(GBUF // 16):
                w = dbuf[k, pl.ds(16 * i, 16)]
                acc0 = acc0 + lax.bitcast_convert_type(
                    lax.shift_left(w, 16), jnp.float32)
                acc1 = acc1 + lax.bitcast_convert_type(
                    jnp.bitwise_and(w, himask), jnp.float32)
            tot0 = jnp.sum(acc0) * INV_S
            tot1 = jnp.sum(acc1) * INV_S

            @pl.when(e + NDEEP < BPW)
            def _():
                issue(e + NDEEP, k)

            res = jnp.where(lane == 0, tot0,
                            jnp.where(lane == 1, tot1, 0.0))
            res = res + b_v[...]
            out_v[pl.ds(C * e, 16)] = res

    pltpu.sync_copy(out_v.at[pl.ds(0, BPW * C)],
                    out_hbm.at[pl.ds(base * C, BPW * C)])


@functools.cache
def _build_sc():
    mesh = plsc.VectorSubcoreMesh(core_axis_name="c", subcore_axis_name="s",
                                  num_cores=NC, num_subcores=NS)

    @functools.partial(
        pl.kernel,
        out_type=jax.ShapeDtypeStruct((B * C,), jnp.float32),
        mesh=mesh,
        compiler_params=pltpu.CompilerParams(needs_layout_passes=False,
                                             use_tc_tiling_on_sc=False),
        scratch_types=[
            pltpu.VMEM((BPW * 2, HPAD), jnp.int32),
            pltpu.VMEM((NDEEP, GBUF), jnp.int32),
            pltpu.VMEM((16,), jnp.float32),
            pltpu.VMEM((BPW * C + 16,), jnp.float32),
        ] + [pltpu.SemaphoreType.DMA] * NDEEP,
    )
    def _gather_pool(ids_hbm, p_hbm, b_hbm, out_hbm, *scratch):
        _sc_body(ids_hbm, p_hbm, b_hbm, out_hbm, *scratch)

    return _gather_pool


def kernel(input_ids, emb_table, head_W, head_b):
    (p,) = _project(head_W, emb_table.T)
    p = jnp.pad(p, (0, 8))
    ids = input_ids.astype(jnp.int32).reshape(B * 2, HALF)
    ids = jnp.pad(ids, ((0, 0), (0, HPAD - HALF)), constant_values=V)
    b16 = jnp.zeros((16,), jnp.float32).at[:C].set(head_b)
    out = _build_sc()(ids, p, b16)
    return out.reshape(B, C)


# speedup vs baseline: 1.9800x; 1.9800x over previous
import functools

import jax
import jax.numpy as jnp
from jax import lax
from jax.experimental import pallas as pl
from jax.experimental.pallas import tpu as pltpu
from jax.experimental.pallas import tpu_sc as plsc

B = 4096
S = 200
D = 64
C = 2
V = 1000000
HALF = S // 2
HPAD = 104
GBUF = 2 * HPAD
NC = 2
NS = 16
NW = NC * NS
BPW = B // NW
INV_S = 1.0 / S
NB = 16384
NDEEP = 16
PSLICE = 62504
VPAD = 16 * PSLICE


def _proj_body(w_ref, t_ref, p_ref):
    m = lax.dot_general(w_ref[...], t_ref[...], (((1,), (0,)), ((), ())),
                        preferred_element_type=jnp.float32)
    r0 = m[0].astype(jnp.bfloat16).astype(jnp.float32)
    r1 = m[1].astype(jnp.bfloat16).astype(jnp.float32)
    b0 = lax.shift_right_logical(lax.bitcast_convert_type(r0, jnp.int32), 16)
    b1 = jnp.bitwise_and(lax.bitcast_convert_type(r1, jnp.int32),
                         jnp.int32(-65536))
    p_ref[...] = jnp.bitwise_or(b0, b1)


def _project(head_W, table_t):
    return pl.pallas_call(
        _proj_body,
        grid=(pl.cdiv(V, NB),),
        in_specs=[
            pl.BlockSpec((C, D), lambda k: (0, 0)),
            pl.BlockSpec((D, NB), lambda k: (0, k)),
        ],
        out_specs=[
            pl.BlockSpec((NB,), lambda k: (k,)),
        ],
        out_shape=[
            jax.ShapeDtypeStruct((V,), jnp.int32),
        ],
    )(head_W, table_t)


def _sc_body(ids_hbm, p_hbm, b_hbm, out_hbm,
             ids_v, dbuf, p_sh, b_v, out_v, *sems):
    wid = lax.axis_index("s") * NC + lax.axis_index("c")
    sid = lax.axis_index("s")
    base = wid * BPW

    pltpu.sync_copy(p_hbm.at[pl.ds(sid * PSLICE, PSLICE)],
                    p_sh.at[pl.ds(sid * PSLICE, PSLICE)])
    pltpu.sync_copy(ids_hbm.at[pl.ds(base * 2, BPW * 2)], ids_v)
    pltpu.sync_copy(b_hbm, b_v)
    plsc.subcore_barrier()

    def issue(e, k):
        pltpu.async_copy(p_sh.at[ids_v.at[2 * e]],
                         dbuf.at[k, pl.ds(0, HPAD)], sems[k])
        pltpu.async_copy(p_sh.at[ids_v.at[2 * e + 1]],
                         dbuf.at[k, pl.ds(HPAD, HPAD)], sems[k])

    def drain(e, k):
        pltpu.make_async_copy(p_sh.at[ids_v.at[2 * e]],
                              dbuf.at[k, pl.ds(0, HPAD)], sems[k]).wait()
        pltpu.make_async_copy(p_sh.at[ids_v.at[2 * e + 1]],
                              dbuf.at[k, pl.ds(HPAD, HPAD)], sems[k]).wait()

    for k in range(NDEEP):
        issue(k, k)

    lane = lax.iota(jnp.int32, 16)
    himask = jnp.full((16,), -65536, jnp.int32)

    @pl.loop(0, BPW, step=NDEEP)
    def _elements(e0):
        for k in range(NDEEP):
            e = e0 + k
            drain(e, k)

            acc0 = jnp.zeros((16,), jnp.float32)
            acc1 = jnp.zeros((16,), jnp.float32)
            for i in range(GBUF // 16):
                w = dbuf[k, pl.ds(16 * i, 16)]
                acc0 = acc0 + lax.bitcast_convert_type(
                    lax.shift_left(w, 16), jnp.float32)
                acc1 = acc1 + lax.bitcast_convert_type(
                    jnp.bitwise_and(w, himask), jnp.float32)
            tot0 = jnp.sum(acc0) * INV_S
            tot1 = jnp.sum(acc1) * INV_S

            @pl.when(e + NDEEP < BPW)
            def _():
                issue(e + NDEEP, k)

            res = jnp.where(lane == 0, tot0,
                            jnp.where(lane == 1, tot1, 0.0))
            res = res + b_v[...]
            out_v[pl.ds(C * e, 16)] = res

    pltpu.sync_copy(out_v.at[pl.ds(0, BPW * C)],
                    out_hbm.at[pl.ds(base * C, BPW * C)])


@functools.cache
def _build_sc():
    mesh = plsc.VectorSubcoreMesh(core_axis_name="c", subcore_axis_name="s",
                                  num_cores=NC, num_subcores=NS)

    @functools.partial(
        pl.kernel,
        out_type=jax.ShapeDtypeStruct((B * C,), jnp.float32),
        mesh=mesh,
        compiler_params=pltpu.CompilerParams(needs_layout_passes=False,
                                             use_tc_tiling_on_sc=False),
        scratch_types=[
            pltpu.VMEM((BPW * 2, HPAD), jnp.int32),
            pltpu.VMEM((NDEEP, GBUF), jnp.int32),
            pltpu.VMEM_SHARED((VPAD,), jnp.int32),
            pltpu.VMEM((16,), jnp.float32),
            pltpu.VMEM((BPW * C + 16,), jnp.float32),
        ] + [pltpu.SemaphoreType.DMA] * NDEEP,
    )
    def _gather_pool(ids_hbm, p_hbm, b_hbm, out_hbm, *scratch):
        _sc_body(ids_hbm, p_hbm, b_hbm, out_hbm, *scratch)

    return _gather_pool


def kernel(input_ids, emb_table, head_W, head_b):
    (p,) = _project(head_W, emb_table.T)
    p = jnp.pad(p, (0, VPAD - V))
    ids = input_ids.astype(jnp.int32).reshape(B * 2, HALF)
    ids = jnp.pad(ids, ((0, 0), (0, HPAD - HALF)), constant_values=V)
    b16 = jnp.zeros((16,), jnp.float32).at[:C].set(head_b)
    out = _build_sc()(ids, p, b16)
    return out.reshape(B, C)


# speedup vs baseline: 2.1303x; 1.0759x over previous
import functools

import jax
import jax.numpy as jnp
from jax import lax
from jax.experimental import pallas as pl
from jax.experimental.pallas import tpu as pltpu
from jax.experimental.pallas import tpu_sc as plsc

B = 4096
S = 200
D = 64
C = 2
V = 1000000
HALF = S // 2
HPAD = 104
GBUF = 2 * HPAD
NC = 2
NS = 16
NW = NC * NS
BPW = B // NW
INV_S = 1.0 / S
NB = 32768
NDEEP = 16
PSLICE = 62504
VPAD = 16 * PSLICE


def _proj_body(w_ref, t_ref, p_ref):
    m = lax.dot_general(w_ref[...], t_ref[...], (((1,), (0,)), ((), ())),
                        preferred_element_type=jnp.float32)
    r0 = m[0].astype(jnp.bfloat16).astype(jnp.float32)
    r1 = m[1].astype(jnp.bfloat16).astype(jnp.float32)
    b0 = lax.shift_right_logical(lax.bitcast_convert_type(r0, jnp.int32), 16)
    b1 = jnp.bitwise_and(lax.bitcast_convert_type(r1, jnp.int32),
                         jnp.int32(-65536))
    p_ref[...] = jnp.bitwise_or(b0, b1)


def _project(head_W, table_t):
    return pl.pallas_call(
        _proj_body,
        grid=(pl.cdiv(V, NB),),
        in_specs=[
            pl.BlockSpec((C, D), lambda k: (0, 0)),
            pl.BlockSpec((D, NB), lambda k: (0, k)),
        ],
        out_specs=[
            pl.BlockSpec((NB,), lambda k: (k,)),
        ],
        out_shape=[
            jax.ShapeDtypeStruct((V,), jnp.int32),
        ],
    )(head_W, table_t)


def _sc_body(ids_hbm, p_hbm, b_hbm, out_hbm,
             ids_v, dbuf, p_sh, b_v, out_v, *sems):
    wid = lax.axis_index("s") * NC + lax.axis_index("c")
    sid = lax.axis_index("s")
    base = wid * BPW

    pltpu.sync_copy(p_hbm.at[pl.ds(sid * PSLICE, PSLICE)],
                    p_sh.at[pl.ds(sid * PSLICE, PSLICE)])
    pltpu.sync_copy(ids_hbm.at[pl.ds(base * 2, BPW * 2)], ids_v)
    pltpu.sync_copy(b_hbm, b_v)
    plsc.subcore_barrier()

    def issue(e, k):
        pltpu.async_copy(p_sh.at[ids_v.at[2 * e]],
                         dbuf.at[k, pl.ds(0, HPAD)], sems[k])
        pltpu.async_copy(p_sh.at[ids_v.at[2 * e + 1]],
                         dbuf.at[k, pl.ds(HPAD, HPAD)], sems[k])

    def drain(e, k):
        pltpu.make_async_copy(p_sh.at[ids_v.at[2 * e]],
                              dbuf.at[k, pl.ds(0, HPAD)], sems[k]).wait()
        pltpu.make_async_copy(p_sh.at[ids_v.at[2 * e + 1]],
                              dbuf.at[k, pl.ds(HPAD, HPAD)], sems[k]).wait()

    for k in range(NDEEP):
        issue(k, k)

    lane = lax.iota(jnp.int32, 16)
    himask = jnp.full((16,), -65536, jnp.int32)

    @pl.loop(0, BPW, step=NDEEP)
    def _elements(e0):
        for k in range(NDEEP):
            e = e0 + k
            drain(e, k)

            acc0 = jnp.zeros((16,), jnp.float32)
            acc1 = jnp.zeros((16,), jnp.float32)
            for i in range(GBUF // 16):
                w = dbuf[k, pl.ds(16 * i, 16)]
                acc0 = acc0 + lax.bitcast_convert_type(
                    lax.shift_left(w, 16), jnp.float32)
                acc1 = acc1 + lax.bitcast_convert_type(
                    jnp.bitwise_and(w, himask), jnp.float32)
            tot0 = jnp.sum(acc0) * INV_S
            tot1 = jnp.sum(acc1) * INV_S

            @pl.when(e + NDEEP < BPW)
            def _():
                issue(e + NDEEP, k)

            res = jnp.where(lane == 0, tot0,
                            jnp.where(lane == 1, tot1, 0.0))
            res = res + b_v[...]
            out_v[pl.ds(C * e, 16)] = res

    pltpu.sync_copy(out_v.at[pl.ds(0, BPW * C)],
                    out_hbm.at[pl.ds(base * C, BPW * C)])


@functools.cache
def _build_sc():
    mesh = plsc.VectorSubcoreMesh(core_axis_name="c", subcore_axis_name="s",
                                  num_cores=NC, num_subcores=NS)

    @functools.partial(
        pl.kernel,
        out_type=jax.ShapeDtypeStruct((B * C,), jnp.float32),
        mesh=mesh,
        compiler_params=pltpu.CompilerParams(needs_layout_passes=False,
                                             use_tc_tiling_on_sc=False),
        scratch_types=[
            pltpu.VMEM((BPW * 2, HPAD), jnp.int32),
            pltpu.VMEM((NDEEP, GBUF), jnp.int32),
            pltpu.VMEM_SHARED((VPAD,), jnp.int32),
            pltpu.VMEM((16,), jnp.float32),
            pltpu.VMEM((BPW * C + 16,), jnp.float32),
        ] + [pltpu.SemaphoreType.DMA] * NDEEP,
    )
    def _gather_pool(ids_hbm, p_hbm, b_hbm, out_hbm, *scratch):
        _sc_body(ids_hbm, p_hbm, b_hbm, out_hbm, *scratch)

    return _gather_pool


def kernel(input_ids, emb_table, head_W, head_b):
    (p,) = _project(head_W, emb_table.T)
    p = jnp.pad(p, (0, VPAD - V))
    ids = input_ids.astype(jnp.int32).reshape(B * 2, HALF)
    ids = jnp.pad(ids, ((0, 0), (0, HPAD - HALF)), constant_values=V)
    b16 = jnp.zeros((16,), jnp.float32).at[:C].set(head_b)
    out = _build_sc()(ids, p, b16)
    return out.reshape(B, C)
